# baseline (speedup 1.0000x reference)
import jax
import jax.numpy as jnp
from jax import lax
from jax.experimental import pallas as pl
from jax.experimental.pallas import tpu as pltpu
from jax.experimental.pallas import tpu_sc as plsc

_THRESH = 0.2
_B = 8
_M = 2048
_N = 2048
_BR = 256
_NRB = _M // _BR


def _stats(x, r):
    bias = jnp.int32(0x3F800000)
    bc = lambda v: lax.bitcast_convert_type(v + bias, jnp.float32)
    unbc = lambda v: lax.bitcast_convert_type(v, jnp.int32) - bias
    rmax = jnp.max(x, axis=2)
    bcmax = jnp.max(x, axis=0)
    lane_i = bc(lax.broadcasted_iota(jnp.int32, x.shape, 2))
    row_i = bc(lax.broadcasted_iota(jnp.int32, x.shape, 0))
    rarg = unbc(jnp.min(jnp.where(x == rmax[:, :, None], lane_i, bc(jnp.int32(_N))),
                        axis=2))
    bcarg = unbc(jnp.min(jnp.where(x == bcmax[None], row_i, bc(jnp.int32(_BR))),
                         axis=0)) + r * _BR
    return rmax, rarg, bcmax, bcarg


_NH = _N // 2


def _phase1_body(xa_ref, xb_ref, max0_ref, idx0_ref, idx1_ref, cmax_s, carg_s):
    r = pl.program_id(0)
    rmax_a, rarg_a, bcmax_a, bcarg_a = _stats(xa_ref[...], r)
    rmax_b, rarg_b, bcmax_b, bcarg_b = _stats(xb_ref[...], r)
    a_wins = rmax_a >= rmax_b
    rmax = jnp.where(a_wins, rmax_a, rmax_b)
    rarg = jnp.where(a_wins, rarg_a, rarg_b + _NH)
    bcmax = jnp.concatenate([bcmax_a, bcmax_b], axis=1)
    bcarg = jnp.concatenate([bcarg_a, bcarg_b], axis=1)
    max0_ref[:, pl.ds(r * _BR, _BR)] = rmax.T
    idx0_ref[:, pl.ds(r * _BR, _BR)] = rarg.T

    @pl.when(r == 0)
    def _():
        cmax_s[...] = bcmax
        carg_s[...] = bcarg

    @pl.when(r > 0)
    def _():
        upd = bcmax > cmax_s[...]
        cmax_s[...] = jnp.where(upd, bcmax, cmax_s[...])
        carg_s[...] = jnp.where(upd, bcarg, carg_s[...])

    @pl.when(r == _NRB - 1)
    def _():
        idx1_ref[...] = carg_s[...]


def _phase1(scores):
    scores_t = jnp.transpose(scores, (1, 0, 2))
    return pl.pallas_call(
        _phase1_body,
        grid=(_NRB,),
        in_specs=[
            pl.BlockSpec((_BR, _B, _NH), lambda r: (r, 0, 0)),
            pl.BlockSpec((_BR, _B, _NH), lambda r: (r, 0, 1)),
        ],
        out_specs=[
            pl.BlockSpec((_B, _M), lambda r: (0, 0)),
            pl.BlockSpec((_B, _M), lambda r: (0, 0)),
            pl.BlockSpec((_B, _N), lambda r: (0, 0)),
        ],
        out_shape=[
            jax.ShapeDtypeStruct((_B, _M), jnp.float32),
            jax.ShapeDtypeStruct((_B, _M), jnp.int32),
            jax.ShapeDtypeStruct((_B, _N), jnp.int32),
        ],
        scratch_shapes=[
            pltpu.VMEM((_B, _N), jnp.float32),
            pltpu.VMEM((_B, _N), jnp.int32),
        ],
    )(scores_t, scores_t)


_L = 16
_QUARTER = _M // 4


def _phase2_body(i0_hbm, i1_hbm, mx_hbm,
                 oi0_hbm, oi1_hbm, om0_hbm, om1_hbm,
                 t_i0, t_i1, t_mx, t_m0, o_i0, o_i1, o_m1):
    wid = lax.axis_index("s") * 2 + lax.axis_index("c")
    b = wid // 4
    q = wid % 4

    pltpu.sync_copy(i0_hbm.at[b], t_i0)
    pltpu.sync_copy(i1_hbm.at[b], t_i1)
    pltpu.sync_copy(mx_hbm.at[b], t_mx)

    def body_a(i, carry):
        off = i * _L
        vi0 = t_i0[pl.ds(off, _L)]
        g = plsc.load_gather(t_i1, [vi0])
        lanes = lax.iota(jnp.int32, _L) + off
        mut0 = g == lanes
        e = jnp.exp(t_mx[pl.ds(off, _L)])
        t_m0[pl.ds(off, _L)] = jnp.where(mut0, e, jnp.float32(0))
        return carry

    lax.fori_loop(0, _M // _L, body_a, 0)

    def body_b(j, carry):
        off = q * _QUARTER + j * _L
        lanes = lax.iota(jnp.int32, _L) + off
        m0 = t_m0[pl.ds(off, _L)]
        vi0 = t_i0[pl.ds(off, _L)]
        o_i0[pl.ds(j * _L, _L)] = jnp.where(m0 > _THRESH, vi0, jnp.int32(-1))
        vi1 = t_i1[pl.ds(off, _L)]
        g1 = plsc.load_gather(t_i0, [vi1])
        mut1 = g1 == lanes
        gm = plsc.load_gather(t_m0, [vi1])
        m1 = jnp.where(mut1, gm, jnp.float32(0))
        o_m1[pl.ds(j * _L, _L)] = m1
        o_i1[pl.ds(j * _L, _L)] = jnp.where(m1 > _THRESH, vi1, jnp.int32(-1))
        return carry

    lax.fori_loop(0, _QUARTER // _L, body_b, 0)

    obase = q * _QUARTER
    pltpu.sync_copy(o_i0, oi0_hbm.at[b, pl.ds(obase, _QUARTER)])
    pltpu.sync_copy(o_i1, oi1_hbm.at[b, pl.ds(obase, _QUARTER)])
    pltpu.sync_copy(t_m0.at[pl.ds(obase, _QUARTER)],
                    om0_hbm.at[b, pl.ds(obase, _QUARTER)])
    pltpu.sync_copy(o_m1, om1_hbm.at[b, pl.ds(obase, _QUARTER)])


def _phase2(i0, i1, mx):
    f32 = jnp.float32
    i32 = jnp.int32
    run = pl.kernel(
        _phase2_body,
        mesh=plsc.VectorSubcoreMesh(core_axis_name="c", subcore_axis_name="s"),
        compiler_params=pltpu.CompilerParams(needs_layout_passes=False),
        out_type=[
            jax.ShapeDtypeStruct((_B, _M), i32),
            jax.ShapeDtypeStruct((_B, _M), i32),
            jax.ShapeDtypeStruct((_B, _M), f32),
            jax.ShapeDtypeStruct((_B, _M), f32),
        ],
        scratch_types=[
            pltpu.VMEM((_M,), i32),
            pltpu.VMEM((_M,), i32),
            pltpu.VMEM((_M,), f32),
            pltpu.VMEM((_M,), f32),
            pltpu.VMEM((_QUARTER,), i32),
            pltpu.VMEM((_QUARTER,), i32),
            pltpu.VMEM((_QUARTER,), f32),
        ],
    )
    return run(i0, i1, mx)


def kernel(scores):
    mx, i0, i1 = _phase1(scores)
    return tuple(_phase2(i0, i1, mx))


# speedup vs baseline: 1.0055x; 1.0055x over previous
import jax
import jax.numpy as jnp
from jax import lax
from jax.experimental import pallas as pl
from jax.experimental.pallas import tpu as pltpu
from jax.experimental.pallas import tpu_sc as plsc

_THRESH = 0.2
_B = 8
_M = 2048
_N = 2048
_BR = 256
_NRB = _M // _BR


def _stats(x, r):
    bias = jnp.int32(0x3F800000)
    bc = lambda v: lax.bitcast_convert_type(v + bias, jnp.float32)
    unbc = lambda v: lax.bitcast_convert_type(v, jnp.int32) - bias
    rmax = jnp.max(x, axis=2)
    bcmax = jnp.max(x, axis=0)
    lane_i = bc(lax.broadcasted_iota(jnp.int32, x.shape, 2))
    row_i = bc(lax.broadcasted_iota(jnp.int32, x.shape, 0))
    rarg = unbc(jnp.min(jnp.where(x == rmax[:, :, None], lane_i, bc(jnp.int32(_N))),
                        axis=2))
    bcarg = unbc(jnp.min(jnp.where(x == bcmax[None], row_i, bc(jnp.int32(_BR))),
                         axis=0)) + r * _BR
    return rmax, rarg, bcmax, bcarg


def _phase1_body(x_ref, max0_ref, idx0_ref, idx1_ref, cmax_s, carg_s):
    r = pl.program_id(0)
    rmax, rarg, bcmax, bcarg = _stats(x_ref[...], r)
    max0_ref[:, pl.ds(r * _BR, _BR)] = rmax.T
    idx0_ref[:, pl.ds(r * _BR, _BR)] = rarg.T

    @pl.when(r == 0)
    def _():
        cmax_s[...] = bcmax
        carg_s[...] = bcarg

    @pl.when(r > 0)
    def _():
        upd = bcmax > cmax_s[...]
        cmax_s[...] = jnp.where(upd, bcmax, cmax_s[...])
        carg_s[...] = jnp.where(upd, bcarg, carg_s[...])

    @pl.when(r == _NRB - 1)
    def _():
        idx1_ref[...] = carg_s[...]


def _phase1(scores):
    scores_t = jnp.transpose(scores, (1, 0, 2))
    return pl.pallas_call(
        _phase1_body,
        grid=(_NRB,),
        in_specs=[pl.BlockSpec((_BR, _B, _N), lambda r: (r, 0, 0))],
        out_specs=[
            pl.BlockSpec((_B, _M), lambda r: (0, 0)),
            pl.BlockSpec((_B, _M), lambda r: (0, 0)),
            pl.BlockSpec((_B, _N), lambda r: (0, 0)),
        ],
        out_shape=[
            jax.ShapeDtypeStruct((_B, _M), jnp.float32),
            jax.ShapeDtypeStruct((_B, _M), jnp.int32),
            jax.ShapeDtypeStruct((_B, _N), jnp.int32),
        ],
        scratch_shapes=[
            pltpu.VMEM((_B, _N), jnp.float32),
            pltpu.VMEM((_B, _N), jnp.int32),
        ],
    )(scores_t)


_L = 16
_QUARTER = _M // 4


def _phase2_body(i0_hbm, i1_hbm, mx_hbm,
                 oi0_hbm, oi1_hbm, om0_hbm, om1_hbm,
                 t_i0, t_i1, t_mx, t_m0, o_i0, o_i1, o_m1):
    wid = lax.axis_index("s") * 2 + lax.axis_index("c")
    b = wid // 4
    q = wid % 4

    pltpu.sync_copy(i0_hbm.at[b], t_i0)
    pltpu.sync_copy(i1_hbm.at[b], t_i1)
    pltpu.sync_copy(mx_hbm.at[b], t_mx)

    def body_a(i, carry):
        off = i * _L
        vi0 = t_i0[pl.ds(off, _L)]
        g = plsc.load_gather(t_i1, [vi0])
        lanes = lax.iota(jnp.int32, _L) + off
        mut0 = g == lanes
        e = jnp.exp(t_mx[pl.ds(off, _L)])
        t_m0[pl.ds(off, _L)] = jnp.where(mut0, e, jnp.float32(0))
        return carry

    lax.fori_loop(0, _M // _L, body_a, 0)

    def body_b(j, carry):
        off = q * _QUARTER + j * _L
        lanes = lax.iota(jnp.int32, _L) + off
        m0 = t_m0[pl.ds(off, _L)]
        vi0 = t_i0[pl.ds(off, _L)]
        o_i0[pl.ds(j * _L, _L)] = jnp.where(m0 > _THRESH, vi0, jnp.int32(-1))
        vi1 = t_i1[pl.ds(off, _L)]
        g1 = plsc.load_gather(t_i0, [vi1])
        mut1 = g1 == lanes
        gm = plsc.load_gather(t_m0, [vi1])
        m1 = jnp.where(mut1, gm, jnp.float32(0))
        o_m1[pl.ds(j * _L, _L)] = m1
        o_i1[pl.ds(j * _L, _L)] = jnp.where(m1 > _THRESH, vi1, jnp.int32(-1))
        return carry

    lax.fori_loop(0, _QUARTER // _L, body_b, 0)

    obase = q * _QUARTER
    pltpu.sync_copy(o_i0, oi0_hbm.at[b, pl.ds(obase, _QUARTER)])
    pltpu.sync_copy(o_i1, oi1_hbm.at[b, pl.ds(obase, _QUARTER)])
    pltpu.sync_copy(t_m0.at[pl.ds(obase, _QUARTER)],
                    om0_hbm.at[b, pl.ds(obase, _QUARTER)])
    pltpu.sync_copy(o_m1, om1_hbm.at[b, pl.ds(obase, _QUARTER)])


def _phase2(i0, i1, mx):
    f32 = jnp.float32
    i32 = jnp.int32
    run = pl.kernel(
        _phase2_body,
        mesh=plsc.VectorSubcoreMesh(core_axis_name="c", subcore_axis_name="s"),
        compiler_params=pltpu.CompilerParams(needs_layout_passes=False),
        out_type=[
            jax.ShapeDtypeStruct((_B, _M), i32),
            jax.ShapeDtypeStruct((_B, _M), i32),
            jax.ShapeDtypeStruct((_B, _M), f32),
            jax.ShapeDtypeStruct((_B, _M), f32),
        ],
        scratch_types=[
            pltpu.VMEM((_M,), i32),
            pltpu.VMEM((_M,), i32),
            pltpu.VMEM((_M,), f32),
            pltpu.VMEM((_M,), f32),
            pltpu.VMEM((_QUARTER,), i32),
            pltpu.VMEM((_QUARTER,), i32),
            pltpu.VMEM((_QUARTER,), f32),
        ],
    )
    return run(i0, i1, mx)


def kernel(scores):
    mx, i0, i1 = _phase1(scores)
    return tuple(_phase2(i0, i1, mx))
